# baseline (speedup 1.0000x reference)
import jax
import jax.numpy as jnp
from jax.experimental import pallas as pl
from jax.experimental.pallas import tpu as pltpu

N = 2048
C = 128
OUT = 128
B = 8
R = 128
NBLK = N // R

_F32 = jnp.float32
_BF16 = jnp.bfloat16


def _body(x0, x1, x2, seg, w00, w10, w01, w11, w21, w12, w22,
          lw0, lw1, lw2, lbs,
          aup0, inc1, inc1t, adn1, aup1, inc2, inc2t, adn2,
          out,
          t00, t10, t01, t11, t21, t12, t22, acc0, acc1, acc2):
    i = pl.program_id(0)

    @pl.when(i == 0)
    def _init():
        x0b = x0[...].astype(_BF16)
        x1b = x1[...].astype(_BF16)
        x2b = x2[...].astype(_BF16)
        t00[...] = jnp.dot(x0b, w00[...].astype(_BF16),
                           preferred_element_type=_F32).astype(_BF16)
        t10[...] = jnp.dot(x1b, w10[...].astype(_BF16),
                           preferred_element_type=_F32).astype(_BF16)
        t01[...] = jnp.dot(x0b, w01[...].astype(_BF16),
                           preferred_element_type=_F32).astype(_BF16)
        t11[...] = jnp.dot(x1b, w11[...].astype(_BF16),
                           preferred_element_type=_F32).astype(_BF16)
        t21[...] = jnp.dot(x2b, w21[...].astype(_BF16),
                           preferred_element_type=_F32).astype(_BF16)
        t12[...] = jnp.dot(x1b, w12[...].astype(_BF16),
                           preferred_element_type=_F32).astype(_BF16)
        t22[...] = jnp.dot(x2b, w22[...].astype(_BF16),
                           preferred_element_type=_F32).astype(_BF16)
        acc0[...] = jnp.zeros((B, C), _F32)
        acc1[...] = jnp.zeros((B, C), _F32)
        acc2[...] = jnp.zeros((B, C), _F32)

    y0 = jax.nn.sigmoid(
        jnp.dot(aup0[...].astype(_BF16), t00[...], preferred_element_type=_F32)
        + jnp.dot(inc1[...].astype(_BF16), t10[...], preferred_element_type=_F32))
    y1 = jax.nn.sigmoid(
        jnp.dot(inc1t[...].astype(_BF16), t01[...], preferred_element_type=_F32)
        + jnp.dot((adn1[...] + aup1[...]).astype(_BF16), t11[...],
                  preferred_element_type=_F32)
        + jnp.dot(inc2[...].astype(_BF16), t21[...], preferred_element_type=_F32))
    y2 = jax.nn.sigmoid(
        jnp.dot(inc2t[...].astype(_BF16), t12[...], preferred_element_type=_F32)
        + jnp.dot(adn2[...].astype(_BF16), t22[...], preferred_element_type=_F32))

    iota = jax.lax.broadcasted_iota(jnp.int32, (B, R), 0)
    oh0 = (iota == seg[0:1, pl.ds(i * R, R)]).astype(_F32)
    oh1 = (iota == seg[1:2, pl.ds(i * R, R)]).astype(_F32)
    oh2 = (iota == seg[2:3, pl.ds(i * R, R)]).astype(_F32)
    acc0[...] += jnp.dot(oh0, y0, preferred_element_type=_F32)
    acc1[...] += jnp.dot(oh1, y1, preferred_element_type=_F32)
    acc2[...] += jnp.dot(oh2, y2, preferred_element_type=_F32)

    @pl.when(i == NBLK - 1)
    def _finalize():
        iota_n = jax.lax.broadcasted_iota(jnp.int32, (B, N), 0)
        c0 = jnp.sum((iota_n == seg[0:1, :]).astype(_F32), axis=1, keepdims=True)
        c1 = jnp.sum((iota_n == seg[1:2, :]).astype(_F32), axis=1, keepdims=True)
        c2 = jnp.sum((iota_n == seg[2:3, :]).astype(_F32), axis=1, keepdims=True)
        m0 = jnp.dot(acc0[...] / jnp.maximum(c0, 1.0), lw0[...],
                     preferred_element_type=_F32)
        m1 = jnp.dot(acc1[...] / jnp.maximum(c1, 1.0), lw1[...],
                     preferred_element_type=_F32)
        m2 = jnp.dot(acc2[...] / jnp.maximum(c2, 1.0), lw2[...],
                     preferred_element_type=_F32)
        out[...] = (m0 + m1 + m2
                    + lbs[0:1, :] + lbs[1:2, :] + lbs[2:3, :]) / 3.0


def _full(shape):
    return pl.BlockSpec(shape, lambda i: (0,) * len(shape))


def kernel(x_0, x_1, x_2, incidence_1, incidence_2, incidence_1_transpose,
           incidence_2_transpose, adjacency_up_0_norm, adjacency_up_1_norm,
           adjacency_down_1_norm, adjacency_down_2_norm, signal_belongings,
           W_0_0, W_1_0, W_0_1, W_1_1, W_2_1, W_1_2, W_2_2,
           lw0, lb0, lw1, lb1, lw2, lb2):
    seg8 = jnp.pad(signal_belongings, ((0, B - 3), (0, 0)))
    lbs = jnp.pad(jnp.stack([lb0, lb1, lb2]), ((0, B - 3), (0, 0)))

    row_spec = pl.BlockSpec((R, N), lambda i: (i, 0))
    grid_spec = pltpu.PrefetchScalarGridSpec(
        num_scalar_prefetch=0,
        grid=(NBLK,),
        in_specs=[
            _full((N, C)), _full((N, C)), _full((N, C)),
            _full((B, N)),
            _full((C, C)), _full((C, C)), _full((C, C)),
            _full((C, C)), _full((C, C)), _full((C, C)),
            _full((C, C)),
            _full((C, OUT)), _full((C, OUT)), _full((C, OUT)),
            _full((B, OUT)),
            row_spec, row_spec, row_spec, row_spec,
            row_spec, row_spec, row_spec, row_spec,
        ],
        out_specs=_full((B, OUT)),
        scratch_shapes=[pltpu.VMEM((N, C), _BF16)] * 7
        + [pltpu.VMEM((B, C), _F32)] * 3,
    )
    return pl.pallas_call(
        _body,
        grid_spec=grid_spec,
        out_shape=jax.ShapeDtypeStruct((B, OUT), _F32),
        compiler_params=pltpu.CompilerParams(
            dimension_semantics=("arbitrary",),
        ),
    )(x_0, x_1, x_2, seg8,
      W_0_0, W_1_0, W_0_1, W_1_1, W_2_1, W_1_2, W_2_2,
      lw0, lw1, lw2, lbs,
      adjacency_up_0_norm, incidence_1, incidence_1_transpose,
      adjacency_down_1_norm, adjacency_up_1_norm, incidence_2,
      incidence_2_transpose, adjacency_down_2_norm)
